# baseline (speedup 1.0000x reference)
import functools

import jax
import jax.numpy as jnp
from jax import lax
from jax.experimental import pallas as pl
from jax.experimental.pallas import tpu as pltpu
from jax.experimental.pallas import tpu_sc as plsc

_INFEATURES = 512
_CAPACITY = 16384
_EPS = 1e-8

_NW = 32
_ROWS_PER_W = _CAPACITY // _NW
_CHUNK = 64
_NCHUNK = _ROWS_PER_W // _CHUNK
_NSLICE = _INFEATURES // 16

def _phase1_body(x_hbm, mem_hbm, d_out, n_out, i_out,
                 xv, b0, b1, pd, pn, od, on, oi, sem0, sem1):
    wid = lax.axis_index("c") * 16 + lax.axis_index("s")
    base_row = wid * _ROWS_PER_W

    pltpu.sync_copy(x_hbm, xv)
    xs = [xv[pl.ds(s * 16, 16)] for s in range(_NSLICE)]

    bufs = (b0, b1)
    sems = (sem0, sem1)
    copies = [None, None]
    copies[0] = pltpu.async_copy(
        mem_hbm.at[pl.ds(base_row, _CHUNK)], bufs[0], sems[0])

    for c in range(_NCHUNK):
        nxt = (c + 1) % 2
        if c + 1 < _NCHUNK:
            copies[nxt] = pltpu.async_copy(
                mem_hbm.at[pl.ds(base_row + (c + 1) * _CHUNK, _CHUNK)],
                bufs[nxt], sems[nxt])
        copies[c % 2].wait()
        buf = bufs[c % 2]

        def row_body(r, _, buf=buf, c=c):
            acc_d = buf[r, pl.ds(0, 16)] * xs[0]
            acc_n = buf[r, pl.ds(0, 16)] * buf[r, pl.ds(0, 16)]
            for s in range(1, _NSLICE):
                m = buf[r, pl.ds(s * 16, 16)]
                acc_d = acc_d + m * xs[s]
                acc_n = acc_n + m * m
            pd[c * _CHUNK + r, pl.ds(0, 16)] = acc_d
            pn[c * _CHUNK + r, pl.ds(0, 16)] = acc_n
            return 0

        lax.fori_loop(0, _CHUNK, row_body, 0)

    def red_body(r, carry):
        bnum, bden, bd, bn, bi = carry
        d = jnp.sum(pd[r, pl.ds(0, 16)])
        n = jnp.sum(pn[r, pl.ds(0, 16)])
        num = d * jnp.abs(d)
        den = jnp.maximum(n, 1e-16)
        pred = num * bden > bnum * den
        return (jnp.where(pred, num, bnum),
                jnp.where(pred, den, bden),
                jnp.where(pred, d, bd),
                jnp.where(pred, n, bn),
                jnp.where(pred, base_row + r, bi))

    init = (jnp.float32(-3.4e38), jnp.float32(1.0), jnp.float32(0.0),
            jnp.float32(1.0), jnp.int32(0))
    _, _, bd, bn, bi = lax.fori_loop(0, _ROWS_PER_W, red_body, init)

    od[...] = jnp.full((16,), bd, jnp.float32)
    on[...] = jnp.full((16,), bn, jnp.float32)
    oi[...] = jnp.full((16,), bi, jnp.int32)
    pltpu.sync_copy(od, d_out.at[wid])
    pltpu.sync_copy(on, n_out.at[wid])
    pltpu.sync_copy(oi, i_out.at[wid])


def _merge_body(x_ref, d_ref, n_ref, i_ref, out_ref):
    x = x_ref[...]
    xx = jnp.sum(x * x)
    d = d_ref[...]
    n = n_ref[...]
    i = i_ref[...]
    g = d * jnp.abs(d) / jnp.maximum(n, 1e-16)
    gm = jnp.max(g)
    cand = jnp.where(g == gm, i, jnp.int32(2**31 - 1))
    bi = jnp.min(cand)
    sel = i == bi
    bd = jnp.max(jnp.where(sel, d, -3.4e38))
    bn = jnp.max(jnp.where(sel, n, -3.4e38))
    val = bd / (jnp.maximum(jnp.sqrt(bn), _EPS) * jnp.maximum(jnp.sqrt(xx), _EPS))
    fi = (lax.broadcasted_iota(jnp.int32, (128, 128), 0) * 128
          + lax.broadcasted_iota(jnp.int32, (128, 128), 1))
    out_ref[...] = jnp.where(fi == bi, val, 0.0)


_merge = pl.pallas_call(
    _merge_body,
    out_shape=jax.ShapeDtypeStruct((128, 128), jnp.float32),
)


@functools.lru_cache(maxsize=None)
def _build_phase1():
    mesh = plsc.VectorSubcoreMesh(core_axis_name="c", subcore_axis_name="s")
    return functools.partial(
        pl.kernel,
        out_type=[
            jax.ShapeDtypeStruct((_NW, 16), jnp.float32),
            jax.ShapeDtypeStruct((_NW, 16), jnp.float32),
            jax.ShapeDtypeStruct((_NW, 16), jnp.int32),
        ],
        mesh=mesh,
        compiler_params=pltpu.CompilerParams(
            needs_layout_passes=False, use_tc_tiling_on_sc=False),
        scratch_types=[
            pltpu.VMEM((_INFEATURES,), jnp.float32),
            pltpu.VMEM((_CHUNK, _INFEATURES), jnp.float32),
            pltpu.VMEM((_CHUNK, _INFEATURES), jnp.float32),
            pltpu.VMEM((_ROWS_PER_W, 16), jnp.float32),
            pltpu.VMEM((_ROWS_PER_W, 16), jnp.float32),
            pltpu.VMEM((16,), jnp.float32),
            pltpu.VMEM((16,), jnp.float32),
            pltpu.VMEM((16,), jnp.int32),
            pltpu.SemaphoreType.DMA,
            pltpu.SemaphoreType.DMA,
        ],
    )(_phase1_body)


@jax.jit
def kernel(x, memory):
    d, n, i = _build_phase1()(x, memory)
    out = _merge(x.reshape(1, _INFEATURES),
                 d.reshape(1, _NW * 16),
                 n.reshape(1, _NW * 16),
                 i.reshape(1, _NW * 16))
    return out.reshape(_CAPACITY)


# speedup vs baseline: 1.4401x; 1.4401x over previous
import functools

import jax
import jax.numpy as jnp
from jax import lax
from jax.experimental import pallas as pl
from jax.experimental.pallas import tpu as pltpu
from jax.experimental.pallas import tpu_sc as plsc

_INFEATURES = 512
_CAPACITY = 16384
_EPS = 1e-8

_NW = 32
_ROWS_PER_W = _CAPACITY // _NW
_CHUNK = 32
_NCHUNK = _ROWS_PER_W // _CHUNK
_NSLICE = _INFEATURES // 16


def _phase1_body(x_hbm, mem_hbm, d_out, n_out, i_out,
                 xv, b0, b1, pd, pn, od, on, oi, sem0, sem1):
    wid = lax.axis_index("c") * 16 + lax.axis_index("s")
    base_row = wid * _ROWS_PER_W

    pltpu.sync_copy(x_hbm, xv)
    xs = [xv[pl.ds(s * 16, 16)] for s in range(_NSLICE)]

    bufs = (b0, b1)
    sems = (sem0, sem1)
    copies = [None, None]
    copies[0] = pltpu.async_copy(
        mem_hbm.at[pl.ds(base_row, _CHUNK)], bufs[0], sems[0])

    for c in range(_NCHUNK):
        nxt = (c + 1) % 2
        if c + 1 < _NCHUNK:
            copies[nxt] = pltpu.async_copy(
                mem_hbm.at[pl.ds(base_row + (c + 1) * _CHUNK, _CHUNK)],
                bufs[nxt], sems[nxt])
        copies[c % 2].wait()
        buf = bufs[c % 2]

        def row_body(r, _, buf=buf, c=c):
            acc_d = buf[r, pl.ds(0, 16)] * xs[0]
            acc_n = buf[r, pl.ds(0, 16)] * buf[r, pl.ds(0, 16)]
            for s in range(1, _NSLICE):
                m = buf[r, pl.ds(s * 16, 16)]
                acc_d = acc_d + m * xs[s]
                acc_n = acc_n + m * m
            pd[pl.ds((c * _CHUNK + r) * 16, 16)] = acc_d
            pn[pl.ds((c * _CHUNK + r) * 16, 16)] = acc_n
            return 0

        lax.fori_loop(0, _CHUNK, row_body, 0)

    def red_body(r, carry):
        bnum, bden, bd, bn, bi = carry
        d = jnp.sum(pd[pl.ds(r * 16, 16)])
        n = jnp.sum(pn[pl.ds(r * 16, 16)])
        num = d * jnp.abs(d)
        den = jnp.maximum(n, 1e-16)
        pred = num * bden > bnum * den
        return (jnp.where(pred, num, bnum),
                jnp.where(pred, den, bden),
                jnp.where(pred, d, bd),
                jnp.where(pred, n, bn),
                jnp.where(pred, base_row + r, bi))

    init = (jnp.float32(-3.4e38), jnp.float32(1.0), jnp.float32(0.0),
            jnp.float32(1.0), jnp.int32(0))
    _, _, bd, bn, bi = lax.fori_loop(0, _ROWS_PER_W, red_body, init)

    od[...] = jnp.full((16,), bd, jnp.float32)
    on[...] = jnp.full((16,), bn, jnp.float32)
    oi[...] = jnp.full((16,), bi, jnp.int32)
    pltpu.sync_copy(od, d_out.at[pl.ds(wid * 16, 16)])
    pltpu.sync_copy(on, n_out.at[pl.ds(wid * 16, 16)])
    pltpu.sync_copy(oi, i_out.at[pl.ds(wid * 16, 16)])


def _merge_body(x_ref, d_ref, n_ref, i_ref, out_ref):
    x = x_ref[...]
    xx = jnp.sum(x * x)
    d = d_ref[...]
    n = n_ref[...]
    i = i_ref[...]
    g = d * jnp.abs(d) / jnp.maximum(n, 1e-16)
    gm = jnp.max(g)
    cand = jnp.where(g == gm, i, jnp.int32(2**31 - 1))
    bi = jnp.min(cand)
    sel = i == bi
    bd = jnp.max(jnp.where(sel, d, -3.4e38))
    bn = jnp.max(jnp.where(sel, n, -3.4e38))
    val = bd / (jnp.maximum(jnp.sqrt(bn), _EPS) * jnp.maximum(jnp.sqrt(xx), _EPS))
    fi = (lax.broadcasted_iota(jnp.int32, (128, 128), 0) * 128
          + lax.broadcasted_iota(jnp.int32, (128, 128), 1))
    out_ref[...] = jnp.where(fi == bi, val, 0.0)


_merge = pl.pallas_call(
    _merge_body,
    out_shape=jax.ShapeDtypeStruct((128, 128), jnp.float32),
)


@functools.lru_cache(maxsize=None)
def _build_phase1():
    mesh = plsc.VectorSubcoreMesh(core_axis_name="c", subcore_axis_name="s")
    return functools.partial(
        pl.kernel,
        out_type=[
            jax.ShapeDtypeStruct((_NW * 16,), jnp.float32),
            jax.ShapeDtypeStruct((_NW * 16,), jnp.float32),
            jax.ShapeDtypeStruct((_NW * 16,), jnp.int32),
        ],
        mesh=mesh,
        compiler_params=pltpu.CompilerParams(needs_layout_passes=False),
        scratch_types=[
            pltpu.VMEM((_INFEATURES,), jnp.float32),
            pltpu.VMEM((_CHUNK, _INFEATURES), jnp.float32),
            pltpu.VMEM((_CHUNK, _INFEATURES), jnp.float32),
            pltpu.VMEM((_ROWS_PER_W * 16,), jnp.float32),
            pltpu.VMEM((_ROWS_PER_W * 16,), jnp.float32),
            pltpu.VMEM((16,), jnp.float32),
            pltpu.VMEM((16,), jnp.float32),
            pltpu.VMEM((16,), jnp.int32),
            pltpu.SemaphoreType.DMA,
            pltpu.SemaphoreType.DMA,
        ],
    )(_phase1_body)


@jax.jit
def kernel(x, memory):
    d, n, i = _build_phase1()(x, memory)
    out = _merge(x.reshape(1, _INFEATURES),
                 d.reshape(1, _NW * 16),
                 n.reshape(1, _NW * 16),
                 i.reshape(1, _NW * 16))
    return out.reshape(_CAPACITY)


# speedup vs baseline: 1.7624x; 1.2238x over previous
import functools

import jax
import jax.numpy as jnp
from jax import lax
from jax.experimental import pallas as pl
from jax.experimental.pallas import tpu as pltpu
from jax.experimental.pallas import tpu_sc as plsc

_INFEATURES = 512
_CAPACITY = 16384
_EPS = 1e-8

_NW = 32
_ROWS_PER_W = _CAPACITY // _NW
_CHUNK = 32
_NCHUNK = _ROWS_PER_W // _CHUNK
_NSLICE = _INFEATURES // 16
_RB = 4


def _phase1_body(x_hbm, mem_hbm, d_out, n_out, i_out,
                 xv, b0, b1, pd, pn, od, on, oi, sem0, sem1):
    wid = lax.axis_index("c") * 16 + lax.axis_index("s")
    base_row = wid * _ROWS_PER_W

    pltpu.sync_copy(x_hbm, xv)
    xs = [xv[pl.ds(s * 16, 16)] for s in range(_NSLICE)]

    bufs = (b0, b1)
    sems = (sem0, sem1)

    def start(c, k):
        pltpu.async_copy(mem_hbm.at[pl.ds(base_row + c * _CHUNK, _CHUNK)],
                         bufs[k], sems[k])

    def wait(k):
        pltpu.make_async_copy(mem_hbm.at[pl.ds(0, _CHUNK)],
                              bufs[k], sems[k]).wait()

    def process(c, k):
        buf = bufs[k]

        def row_body(rb, _):
            r = rb * _RB
            accs = []
            for q in range(_RB):
                m = buf[r + q, pl.ds(0, 16)]
                accs.append([m * xs[0], m * m])
            for s in range(1, _NSLICE):
                xv_s = xs[s]
                for q in range(_RB):
                    m = buf[r + q, pl.ds(s * 16, 16)]
                    accs[q][0] = accs[q][0] + m * xv_s
                    accs[q][1] = accs[q][1] + m * m
            for q in range(_RB):
                pd[pl.ds((c * _CHUNK + r + q) * 16, 16)] = accs[q][0]
                pn[pl.ds((c * _CHUNK + r + q) * 16, 16)] = accs[q][1]
            return 0

        lax.fori_loop(0, _CHUNK // _RB, row_body, 0)

    start(0, 0)
    start(1, 1)

    def chunk_body(j, _):
        c = j * 2
        wait(0)
        process(c, 0)
        start(c + 2, 0)
        wait(1)
        process(c + 1, 1)
        start(c + 3, 1)
        return 0

    lax.fori_loop(0, _NCHUNK // 2 - 1, chunk_body, 0)
    wait(0)
    process(_NCHUNK - 2, 0)
    wait(1)
    process(_NCHUNK - 1, 1)

    def red_body(r, carry):
        bnum, bden, bd, bn, bi = carry
        d = jnp.sum(pd[pl.ds(r * 16, 16)])
        n = jnp.sum(pn[pl.ds(r * 16, 16)])
        num = d * jnp.abs(d)
        den = jnp.maximum(n, 1e-16)
        pred = num * bden > bnum * den
        return (jnp.where(pred, num, bnum),
                jnp.where(pred, den, bden),
                jnp.where(pred, d, bd),
                jnp.where(pred, n, bn),
                jnp.where(pred, base_row + r, bi))

    init = (jnp.float32(-3.4e38), jnp.float32(1.0), jnp.float32(0.0),
            jnp.float32(1.0), jnp.int32(0))
    _, _, bd, bn, bi = lax.fori_loop(0, _ROWS_PER_W, red_body, init)

    od[...] = jnp.full((16,), bd, jnp.float32)
    on[...] = jnp.full((16,), bn, jnp.float32)
    oi[...] = jnp.full((16,), bi, jnp.int32)
    pltpu.sync_copy(od, d_out.at[pl.ds(wid * 16, 16)])
    pltpu.sync_copy(on, n_out.at[pl.ds(wid * 16, 16)])
    pltpu.sync_copy(oi, i_out.at[pl.ds(wid * 16, 16)])


def _merge_body(x_ref, d_ref, n_ref, i_ref, out_ref):
    x = x_ref[...]
    xx = jnp.sum(x * x)
    d = d_ref[...]
    n = n_ref[...]
    i = i_ref[...]
    g = d * jnp.abs(d) / jnp.maximum(n, 1e-16)
    gm = jnp.max(g)
    cand = jnp.where(g == gm, i, jnp.int32(2**31 - 1))
    bi = jnp.min(cand)
    sel = i == bi
    bd = jnp.max(jnp.where(sel, d, -3.4e38))
    bn = jnp.max(jnp.where(sel, n, -3.4e38))
    val = bd / (jnp.maximum(jnp.sqrt(bn), _EPS) * jnp.maximum(jnp.sqrt(xx), _EPS))
    fi = lax.broadcasted_iota(jnp.int32, (_CAPACITY,), 0)
    out_ref[...] = jnp.where(fi == bi, val, 0.0)


_merge = pl.pallas_call(
    _merge_body,
    out_shape=jax.ShapeDtypeStruct((_CAPACITY,), jnp.float32),
)


@functools.lru_cache(maxsize=None)
def _build_phase1():
    mesh = plsc.VectorSubcoreMesh(core_axis_name="c", subcore_axis_name="s")
    return functools.partial(
        pl.kernel,
        out_type=[
            jax.ShapeDtypeStruct((_NW * 16,), jnp.float32),
            jax.ShapeDtypeStruct((_NW * 16,), jnp.float32),
            jax.ShapeDtypeStruct((_NW * 16,), jnp.int32),
        ],
        mesh=mesh,
        compiler_params=pltpu.CompilerParams(needs_layout_passes=False),
        scratch_types=[
            pltpu.VMEM((_INFEATURES,), jnp.float32),
            pltpu.VMEM((_CHUNK, _INFEATURES), jnp.float32),
            pltpu.VMEM((_CHUNK, _INFEATURES), jnp.float32),
            pltpu.VMEM((_ROWS_PER_W * 16,), jnp.float32),
            pltpu.VMEM((_ROWS_PER_W * 16,), jnp.float32),
            pltpu.VMEM((16,), jnp.float32),
            pltpu.VMEM((16,), jnp.float32),
            pltpu.VMEM((16,), jnp.int32),
            pltpu.SemaphoreType.DMA,
            pltpu.SemaphoreType.DMA,
        ],
    )(_phase1_body)


@jax.jit
def kernel(x, memory):
    d, n, i = _build_phase1()(x, memory)
    return _merge(x, d, n, i)
